# baseline (speedup 1.0000x reference)
import functools

import jax
import jax.numpy as jnp
from jax import lax
from jax.experimental import pallas as pl
from jax.experimental.pallas import tpu as pltpu
from jax.experimental.pallas import tpu_sc as plsc

BATCH = 16384
DIM = 64
NC, NS = 2, 16
NW = NC * NS
B_PER_W = BATCH // NW
CH = 128
NCH = B_PER_W // CH


TW = 4096


def _xpose_body(x1_ref, x2_ref, o_ref):
    o_ref[:, :DIM] = x1_ref[...].astype(jnp.bfloat16).T
    o_ref[:, DIM:] = x2_ref[...].astype(jnp.bfloat16).T


def _tc_transpose_pack(ta_T, tb_T):
    n = ta_T.shape[1]
    return pl.pallas_call(
        _xpose_body,
        grid=(pl.cdiv(n, TW),),
        in_specs=[
            pl.BlockSpec((DIM, TW), lambda i: (0, i)),
            pl.BlockSpec((DIM, TW), lambda i: (0, i)),
        ],
        out_specs=pl.BlockSpec((TW, 2 * DIM), lambda i: (i, 0)),
        out_shape=jax.ShapeDtypeStruct((n, 2 * DIM), jnp.bfloat16),
    )(ta_T, tb_T)


def _sc_gather2(u2, i2, p_user, p_item):
    mesh = plsc.VectorSubcoreMesh(core_axis_name="c", subcore_axis_name="s")
    row_t = jax.ShapeDtypeStruct((BATCH, 2 * DIM), jnp.bfloat16)

    @functools.partial(
        pl.kernel,
        out_type=(row_t, row_t),
        mesh=mesh,
        compiler_params=pltpu.CompilerParams(use_tc_tiling_on_sc=False),
        scratch_types=[
            pltpu.VMEM((NCH, CH), jnp.int32),
            pltpu.VMEM((NCH, CH), jnp.int32),
            pltpu.VMEM((CH, 2 * DIM), jnp.bfloat16),
            pltpu.VMEM((CH, 2 * DIM), jnp.bfloat16),
            pltpu.SemaphoreType.DMA,
            pltpu.SemaphoreType.DMA,
            pltpu.SemaphoreType.DMA,
            pltpu.SemaphoreType.DMA,
        ],
    )
    def k(u_hbm, i_hbm, pu_hbm, pi_hbm, o_u, o_i,
          uidx, iidx, rows_a, rows_b, sem_a, sem_b, sem_wa, sem_wb):
        wid = lax.axis_index("s") * NC + lax.axis_index("c")
        base = wid * B_PER_W

        pltpu.sync_copy(u_hbm.at[pl.ds(wid * NCH, NCH)], uidx)
        pltpu.sync_copy(i_hbm.at[pl.ds(wid * NCH, NCH)], iidx)

        work = []
        for table, idx, out in ((pu_hbm, uidx, o_u), (pi_hbm, iidx, o_i)):
            for c in range(NCH):
                work.append((table, idx, c, out))

        bufs = ((rows_a, sem_a, sem_wa), (rows_b, sem_b, sem_wb))
        pending_w = [None, None]
        for n, (table, idx, c, out) in enumerate(work):
            rows, sem_g, sem_w = bufs[n % 2]
            if pending_w[n % 2] is not None:
                pending_w[n % 2].wait()
            g = pltpu.async_copy(table.at[idx.at[c]], rows, sem_g)
            g.wait()
            pending_w[n % 2] = pltpu.async_copy(
                rows, out.at[pl.ds(base + c * CH, CH)], sem_w)
        for w in pending_w:
            if w is not None:
                w.wait()

    return k(u2, i2, p_user, p_item)


def _mm(a, b):
    return lax.dot_general(a, b, (((1,), (0,)), ((), ())),
                           preferred_element_type=jnp.float32)


BB = 2048


def _dense_body(gu, gi, w0u, w0i, b0, w1, b1, w2, b2, wfm, wfh, bf, o):
    gub = gu[...].astype(jnp.float32)
    gib = gi[...].astype(jnp.float32)
    xmfu = gub[:, :DIM]
    xmlu = gub[:, DIM:]
    xmfi = gib[:, :DIM]
    xmli = gib[:, DIM:]

    h = _mm(xmlu, w0u[...]) + _mm(xmli, w0i[...]) + b0[...]
    h = jnp.maximum(h, 0.0)
    h = jnp.maximum(_mm(h, w1[...]) + b1[...], 0.0)
    h = jnp.maximum(_mm(h, w2[...]) + b2[...], 0.0)
    xmf = xmfu * xmfi
    o[...] = _mm(xmf, wfm[...]) + _mm(h, wfh[...]) + bf[...]


def _tc_dense(gu, gi, w0u, w0i, b0, w1, b1, w2, b2, wfm, wfh, bf):
    bspec = lambda shape: pl.BlockSpec(shape, lambda i: (i, 0))
    wspec = lambda shape: pl.BlockSpec(shape, lambda i: (0, 0))
    return pl.pallas_call(
        _dense_body,
        grid=(BATCH // BB,),
        in_specs=[
            bspec((BB, 2 * DIM)), bspec((BB, 2 * DIM)),
            wspec((DIM, 64)), wspec((DIM, 64)), wspec((1, 64)),
            wspec((64, 32)), wspec((1, 32)),
            wspec((32, 16)), wspec((1, 16)),
            wspec((DIM, 1)), wspec((16, 1)), wspec((1, 1)),
        ],
        out_specs=pl.BlockSpec((BB, 1), lambda i: (i, 0)),
        out_shape=jax.ShapeDtypeStruct((BATCH, 1), jnp.float32),
    )(gu, gi, w0u, w0i, b0, w1, b1, w2, b2, wfm, wfh, bf)


def kernel(user, item, mf_user_embed, mf_item_embed, mlp_user_embed,
           mlp_item_embed, W0, b0, W1, b1, W2, b2, Wf, bf):
    user = user.astype(jnp.int32)
    item = item.astype(jnp.int32)

    p_user = _tc_transpose_pack(mf_user_embed.T, mlp_user_embed.T)
    p_item = _tc_transpose_pack(mf_item_embed.T, mlp_item_embed.T)

    u2 = user.reshape(BATCH // CH, CH)
    i2 = item.reshape(BATCH // CH, CH)
    gu, gi = _sc_gather2(u2, i2, p_user, p_item)

    w0t = W0.T
    wft = Wf.T
    out = _tc_dense(
        gu, gi,
        w0t[:DIM], w0t[DIM:], b0.reshape(1, -1),
        W1.T, b1.reshape(1, -1),
        W2.T, b2.reshape(1, -1),
        wft[:DIM], wft[DIM:], bf.reshape(1, 1))
    return out


# speedup vs baseline: 2.3688x; 2.3688x over previous
import functools

import jax
import jax.numpy as jnp
from jax import lax
from jax.experimental import pallas as pl
from jax.experimental.pallas import tpu as pltpu
from jax.experimental.pallas import tpu_sc as plsc

BATCH = 16384
DIM = 64
NC, NS = 2, 16
NW = NC * NS
B_PER_W = BATCH // NW
CH = 128
NCH = B_PER_W // CH


TW = 4096


def _xpose_body(x1_ref, x2_ref, o_ref):
    o_ref[...] = jnp.concatenate([x1_ref[...].T, x2_ref[...].T], axis=1)


def _tc_transpose_pack(ta_T, tb_T):
    n = ta_T.shape[1]
    return pl.pallas_call(
        _xpose_body,
        grid=(pl.cdiv(n, TW),),
        in_specs=[
            pl.BlockSpec((DIM, TW), lambda i: (0, i)),
            pl.BlockSpec((DIM, TW), lambda i: (0, i)),
        ],
        out_specs=pl.BlockSpec((TW, 2 * DIM), lambda i: (i, 0)),
        out_shape=jax.ShapeDtypeStruct((n, 2 * DIM), jnp.float32),
    )(ta_T, tb_T)


def _sc_gather2(u2, i2, p_user, p_item):
    mesh = plsc.VectorSubcoreMesh(core_axis_name="c", subcore_axis_name="s")
    row_t = jax.ShapeDtypeStruct((BATCH, 2 * DIM), jnp.float32)

    @functools.partial(
        pl.kernel,
        out_type=(row_t, row_t),
        mesh=mesh,
        compiler_params=pltpu.CompilerParams(use_tc_tiling_on_sc=False),
        scratch_types=[
            pltpu.VMEM((NCH, CH), jnp.int32),
            pltpu.VMEM((NCH, CH), jnp.int32),
            pltpu.VMEM((CH, 2 * DIM), jnp.float32),
            pltpu.VMEM((CH, 2 * DIM), jnp.float32),
            pltpu.SemaphoreType.DMA,
            pltpu.SemaphoreType.DMA,
            pltpu.SemaphoreType.DMA,
            pltpu.SemaphoreType.DMA,
        ],
    )
    def k(u_hbm, i_hbm, pu_hbm, pi_hbm, o_u, o_i,
          uidx, iidx, rows_a, rows_b, sem_a, sem_b, sem_wa, sem_wb):
        wid = lax.axis_index("s") * NC + lax.axis_index("c")
        base = wid * B_PER_W

        pltpu.sync_copy(u_hbm.at[pl.ds(wid * NCH, NCH)], uidx)
        pltpu.sync_copy(i_hbm.at[pl.ds(wid * NCH, NCH)], iidx)

        work = []
        for table, idx, out in ((pu_hbm, uidx, o_u), (pi_hbm, iidx, o_i)):
            for c in range(NCH):
                work.append((table, idx, c, out))

        bufs = ((rows_a, sem_a, sem_wa), (rows_b, sem_b, sem_wb))
        pending_w = [None, None]
        for n, (table, idx, c, out) in enumerate(work):
            rows, sem_g, sem_w = bufs[n % 2]
            if pending_w[n % 2] is not None:
                pending_w[n % 2].wait()
            g = pltpu.async_copy(table.at[idx.at[c]], rows, sem_g)
            g.wait()
            pending_w[n % 2] = pltpu.async_copy(
                rows, out.at[pl.ds(base + c * CH, CH)], sem_w)
        for w in pending_w:
            if w is not None:
                w.wait()

    return k(u2, i2, p_user, p_item)


def _mm(a, b):
    return lax.dot_general(a, b, (((1,), (0,)), ((), ())),
                           preferred_element_type=jnp.float32)


BB = 2048


def _dense_body(gu, gi, w0u, w0i, b0, w1, b1, w2, b2, wfm, wfh, bf, o):
    gub = gu[...]
    gib = gi[...]
    xmfu = gub[:, :DIM]
    xmlu = gub[:, DIM:]
    xmfi = gib[:, :DIM]
    xmli = gib[:, DIM:]

    h = _mm(xmlu, w0u[...]) + _mm(xmli, w0i[...]) + b0[...]
    h = jnp.maximum(h, 0.0)
    h = jnp.maximum(_mm(h, w1[...]) + b1[...], 0.0)
    h = jnp.maximum(_mm(h, w2[...]) + b2[...], 0.0)
    xmf = xmfu * xmfi
    o[...] = _mm(xmf, wfm[...]) + _mm(h, wfh[...]) + bf[...]


def _tc_dense(gu, gi, w0u, w0i, b0, w1, b1, w2, b2, wfm, wfh, bf):
    bspec = lambda shape: pl.BlockSpec(shape, lambda i: (i, 0))
    wspec = lambda shape: pl.BlockSpec(shape, lambda i: (0, 0))
    return pl.pallas_call(
        _dense_body,
        grid=(BATCH // BB,),
        in_specs=[
            bspec((BB, 2 * DIM)), bspec((BB, 2 * DIM)),
            wspec((DIM, 64)), wspec((DIM, 64)), wspec((1, 64)),
            wspec((64, 32)), wspec((1, 32)),
            wspec((32, 16)), wspec((1, 16)),
            wspec((DIM, 1)), wspec((16, 1)), wspec((1, 1)),
        ],
        out_specs=pl.BlockSpec((BB, 1), lambda i: (i, 0)),
        out_shape=jax.ShapeDtypeStruct((BATCH, 1), jnp.float32),
    )(gu, gi, w0u, w0i, b0, w1, b1, w2, b2, wfm, wfh, bf)


def kernel(user, item, mf_user_embed, mf_item_embed, mlp_user_embed,
           mlp_item_embed, W0, b0, W1, b1, W2, b2, Wf, bf):
    user = user.astype(jnp.int32)
    item = item.astype(jnp.int32)

    p_user = _tc_transpose_pack(mf_user_embed.T, mlp_user_embed.T)
    p_item = _tc_transpose_pack(mf_item_embed.T, mlp_item_embed.T)

    u2 = user.reshape(BATCH // CH, CH)
    i2 = item.reshape(BATCH // CH, CH)
    gu, gi = _sc_gather2(u2, i2, p_user, p_item)

    w0t = W0.T
    wft = Wf.T
    out = _tc_dense(
        gu, gi,
        w0t[:DIM], w0t[DIM:], b0.reshape(1, -1),
        W1.T, b1.reshape(1, -1),
        W2.T, b2.reshape(1, -1),
        wft[:DIM], wft[DIM:], bf.reshape(1, 1))
    return out


# speedup vs baseline: 3.5584x; 1.5022x over previous
import functools

import jax
import jax.numpy as jnp
from jax import lax
from jax.experimental import pallas as pl
from jax.experimental.pallas import tpu as pltpu
from jax.experimental.pallas import tpu_sc as plsc

BATCH = 16384
DIM = 64
NC, NS = 2, 16
NW = NC * NS
B_PER_W = BATCH // NW
CH = 128
NCH = B_PER_W // CH

TW = 4096


def _pack2(a_ref, b_ref):
    a = lax.bitcast_convert_type(
        a_ref[...].astype(jnp.bfloat16).T, jnp.uint16).astype(jnp.uint32)
    b = lax.bitcast_convert_type(
        b_ref[...].astype(jnp.bfloat16).T, jnp.uint16).astype(jnp.uint32)
    return (a << 16) | b


def _xpose_body(a_lo, b_lo, a_hi, b_hi, o_ref):
    w = jnp.concatenate([_pack2(a_lo, b_lo), _pack2(a_hi, b_hi)], axis=1)
    o_ref[...] = lax.bitcast_convert_type(w, jnp.int32)


def _tc_transpose_pack(ta_T, tb_T):
    n = ta_T.shape[1]
    grid = pl.cdiv(n, 2 * TW)
    max_blk = pl.cdiv(n, TW) - 1
    inspec = lambda f: pl.BlockSpec((DIM, TW), f)
    return pl.pallas_call(
        _xpose_body,
        grid=(grid,),
        in_specs=[
            inspec(lambda i: (0, 2 * i)),
            inspec(lambda i: (0, 2 * i)),
            inspec(lambda i: (0, jnp.minimum(2 * i + 1, max_blk))),
            inspec(lambda i: (0, jnp.minimum(2 * i + 1, max_blk))),
        ],
        out_specs=pl.BlockSpec((TW, 2 * DIM), lambda i: (i, 0)),
        out_shape=jax.ShapeDtypeStruct((grid * TW, 2 * DIM), jnp.int32),
    )(ta_T, tb_T, ta_T, tb_T)


def _sc_gather2(u2, i2, p_user, p_item):
    mesh = plsc.VectorSubcoreMesh(core_axis_name="c", subcore_axis_name="s")
    row_t = jax.ShapeDtypeStruct((BATCH, 2 * DIM), jnp.int32)

    @functools.partial(
        pl.kernel,
        out_type=(row_t, row_t),
        mesh=mesh,
        compiler_params=pltpu.CompilerParams(use_tc_tiling_on_sc=False),
        scratch_types=[
            pltpu.VMEM((NCH, CH), jnp.int32),
            pltpu.VMEM((NCH, CH), jnp.int32),
            pltpu.VMEM((CH, 2 * DIM), jnp.int32),
            pltpu.VMEM((CH, 2 * DIM), jnp.int32),
            pltpu.SemaphoreType.DMA,
            pltpu.SemaphoreType.DMA,
            pltpu.SemaphoreType.DMA,
            pltpu.SemaphoreType.DMA,
        ],
    )
    def k(u_hbm, i_hbm, pu_hbm, pi_hbm, o_u, o_i,
          uidx, iidx, rows_a, rows_b, sem_a, sem_b, sem_wa, sem_wb):
        wid = lax.axis_index("s") * NC + lax.axis_index("c")
        base = wid * B_PER_W

        pltpu.sync_copy(u_hbm.at[pl.ds(wid * NCH, NCH)], uidx)
        pltpu.sync_copy(i_hbm.at[pl.ds(wid * NCH, NCH)], iidx)

        work = []
        for table, idx, out in ((pu_hbm, uidx, o_u), (pi_hbm, iidx, o_i)):
            for c in range(NCH):
                work.append((table, idx, c, out))

        bufs = ((rows_a, sem_a, sem_wa), (rows_b, sem_b, sem_wb))
        pending_w = [None, None]
        for n, (table, idx, c, out) in enumerate(work):
            rows, sem_g, sem_w = bufs[n % 2]
            if pending_w[n % 2] is not None:
                pending_w[n % 2].wait()
            g = pltpu.async_copy(table.at[idx.at[c]], rows, sem_g)
            g.wait()
            pending_w[n % 2] = pltpu.async_copy(
                rows, out.at[pl.ds(base + c * CH, CH)], sem_w)
        for w in pending_w:
            if w is not None:
                w.wait()

    return k(u2, i2, p_user, p_item)


def _mm(a, b):
    return lax.dot_general(a, b, (((1,), (0,)), ((), ())),
                           preferred_element_type=jnp.float32)


BB = 2048

def _unpack(words, par):
    sel = jnp.where(par > 0.5, words[:, DIM:], words[:, :DIM])
    u = lax.bitcast_convert_type(sel, jnp.uint32)
    mf = lax.bitcast_convert_type((u >> 16) << 16, jnp.float32)
    ml = lax.bitcast_convert_type(u << 16, jnp.float32)
    return mf, ml


def _dense_body(gu, gi, upar, ipar,
                w0u, w0i, b0, w1, b1, w2, b2, wfm, wfh, bf, o):
    xmfu, xmlu = _unpack(gu[...], upar[...])
    xmfi, xmli = _unpack(gi[...], ipar[...])

    h = _mm(xmlu, w0u[...]) + _mm(xmli, w0i[...]) + b0[...]
    h = jnp.maximum(h, 0.0)
    h = jnp.maximum(_mm(h, w1[...]) + b1[...], 0.0)
    h = jnp.maximum(_mm(h, w2[...]) + b2[...], 0.0)
    xmf = xmfu * xmfi
    o[...] = _mm(xmf, wfm[...]) + _mm(h, wfh[...]) + bf[...]


def _tc_dense(gu, gi, upar, ipar, w0u, w0i, b0, w1, b1, w2, b2, wfm, wfh, bf):
    bspec = lambda shape: pl.BlockSpec(shape, lambda i: (i, 0))
    wspec = lambda shape: pl.BlockSpec(shape, lambda i: (0, 0))
    return pl.pallas_call(
        _dense_body,
        grid=(BATCH // BB,),
        in_specs=[
            bspec((BB, 2 * DIM)), bspec((BB, 2 * DIM)),
            bspec((BB, 1)), bspec((BB, 1)),
            wspec((DIM, 64)), wspec((DIM, 64)), wspec((1, 64)),
            wspec((64, 32)), wspec((1, 32)),
            wspec((32, 16)), wspec((1, 16)),
            wspec((DIM, 1)), wspec((16, 1)), wspec((1, 1)),
        ],
        out_specs=pl.BlockSpec((BB, 1), lambda i: (i, 0)),
        out_shape=jax.ShapeDtypeStruct((BATCH, 1), jnp.float32),
    )(gu, gi, upar, ipar, w0u, w0i, b0, w1, b1, w2, b2, wfm, wfh, bf)


def kernel(user, item, mf_user_embed, mf_item_embed, mlp_user_embed,
           mlp_item_embed, W0, b0, W1, b1, W2, b2, Wf, bf):
    user = user.astype(jnp.int32)
    item = item.astype(jnp.int32)

    p_user = _tc_transpose_pack(mf_user_embed.T, mlp_user_embed.T)
    p_item = _tc_transpose_pack(mf_item_embed.T, mlp_item_embed.T)

    def rowmap(r):
        w = r // TW
        return (w // 2) * TW + r % TW, (w % 2).astype(jnp.float32)

    u2, upar = rowmap(user)
    i2, ipar = rowmap(item)
    gu, gi = _sc_gather2(u2.reshape(BATCH // CH, CH),
                         i2.reshape(BATCH // CH, CH), p_user, p_item)

    w0t = W0.T
    wft = Wf.T
    out = _tc_dense(
        gu, gi, upar.reshape(BATCH, 1), ipar.reshape(BATCH, 1),
        w0t[:DIM], w0t[DIM:], b0.reshape(1, -1),
        W1.T, b1.reshape(1, -1),
        W2.T, b2.reshape(1, -1),
        wft[:DIM], wft[DIM:], bf.reshape(1, 1))
    return out


# speedup vs baseline: 3.5643x; 1.0016x over previous
import functools

import jax
import jax.numpy as jnp
from jax import lax
from jax.experimental import pallas as pl
from jax.experimental.pallas import tpu as pltpu
from jax.experimental.pallas import tpu_sc as plsc

BATCH = 16384
DIM = 64
NC, NS = 2, 16
NW = NC * NS
B_PER_W = BATCH // NW
CH = 128
NCH = B_PER_W // CH

TW = 4096


def _pack2(a, b):
    au = lax.bitcast_convert_type(
        a.astype(jnp.bfloat16).T, jnp.uint16).astype(jnp.uint32)
    bu = lax.bitcast_convert_type(
        b.astype(jnp.bfloat16).T, jnp.uint16).astype(jnp.uint32)
    return (au << 16) | bu


def _xpose_body(a_ref, b_ref, o_ref):
    a = a_ref[...]
    b = b_ref[...]
    w = jnp.concatenate([_pack2(a[:, :TW], b[:, :TW]),
                         _pack2(a[:, TW:], b[:, TW:])], axis=1)
    o_ref[...] = lax.bitcast_convert_type(w, jnp.int32)


def _tc_transpose_pack(ta_T, tb_T):
    n = ta_T.shape[1]
    grid = pl.cdiv(n, 2 * TW)
    return pl.pallas_call(
        _xpose_body,
        grid=(grid,),
        in_specs=[
            pl.BlockSpec((DIM, 2 * TW), lambda i: (0, i)),
            pl.BlockSpec((DIM, 2 * TW), lambda i: (0, i)),
        ],
        out_specs=pl.BlockSpec((TW, 2 * DIM), lambda i: (i, 0)),
        out_shape=jax.ShapeDtypeStruct((grid * TW, 2 * DIM), jnp.int32),
    )(ta_T, tb_T)


def _sc_gather2(u2, i2, p_user, p_item):
    mesh = plsc.VectorSubcoreMesh(core_axis_name="c", subcore_axis_name="s")
    row_t = jax.ShapeDtypeStruct((BATCH, 2 * DIM), jnp.int32)

    @functools.partial(
        pl.kernel,
        out_type=(row_t, row_t),
        mesh=mesh,
        compiler_params=pltpu.CompilerParams(use_tc_tiling_on_sc=False),
        scratch_types=[
            pltpu.VMEM((NCH, CH), jnp.int32),
            pltpu.VMEM((NCH, CH), jnp.int32),
            pltpu.VMEM((CH, 2 * DIM), jnp.int32),
            pltpu.VMEM((CH, 2 * DIM), jnp.int32),
            pltpu.SemaphoreType.DMA,
            pltpu.SemaphoreType.DMA,
            pltpu.SemaphoreType.DMA,
            pltpu.SemaphoreType.DMA,
        ],
    )
    def k(u_hbm, i_hbm, pu_hbm, pi_hbm, o_u, o_i,
          uidx, iidx, rows_a, rows_b, sem_a, sem_b, sem_wa, sem_wb):
        wid = lax.axis_index("s") * NC + lax.axis_index("c")
        base = wid * B_PER_W

        pltpu.sync_copy(u_hbm.at[pl.ds(wid * NCH, NCH)], uidx)
        pltpu.sync_copy(i_hbm.at[pl.ds(wid * NCH, NCH)], iidx)

        work = []
        for table, idx, out in ((pu_hbm, uidx, o_u), (pi_hbm, iidx, o_i)):
            for c in range(NCH):
                work.append((table, idx, c, out))

        bufs = ((rows_a, sem_a, sem_wa), (rows_b, sem_b, sem_wb))
        pending_w = [None, None]
        for n, (table, idx, c, out) in enumerate(work):
            rows, sem_g, sem_w = bufs[n % 2]
            if pending_w[n % 2] is not None:
                pending_w[n % 2].wait()
            g = pltpu.async_copy(table.at[idx.at[c]], rows, sem_g)
            g.wait()
            pending_w[n % 2] = pltpu.async_copy(
                rows, out.at[pl.ds(base + c * CH, CH)], sem_w)
        for w in pending_w:
            if w is not None:
                w.wait()

    return k(u2, i2, p_user, p_item)


def _mm(a, b):
    return lax.dot_general(a, b, (((1,), (0,)), ((), ())),
                           preferred_element_type=jnp.float32)


BB = 2048

def _unpack(words, par):
    sel = jnp.where(par > 0.5, words[:, DIM:], words[:, :DIM])
    u = lax.bitcast_convert_type(sel, jnp.uint32)
    mf = lax.bitcast_convert_type((u >> 16) << 16, jnp.float32)
    ml = lax.bitcast_convert_type(u << 16, jnp.float32)
    return mf, ml


def _dense_body(gu, gi, upar, ipar,
                w0u, w0i, b0, w1, b1, w2, b2, wfm, wfh, bf, o):
    xmfu, xmlu = _unpack(gu[...], upar[...])
    xmfi, xmli = _unpack(gi[...], ipar[...])

    h = _mm(xmlu, w0u[...]) + _mm(xmli, w0i[...]) + b0[...]
    h = jnp.maximum(h, 0.0)
    h = jnp.maximum(_mm(h, w1[...]) + b1[...], 0.0)
    h = jnp.maximum(_mm(h, w2[...]) + b2[...], 0.0)
    xmf = xmfu * xmfi
    o[...] = _mm(xmf, wfm[...]) + _mm(h, wfh[...]) + bf[...]


def _tc_dense(gu, gi, upar, ipar, w0u, w0i, b0, w1, b1, w2, b2, wfm, wfh, bf):
    bspec = lambda shape: pl.BlockSpec(shape, lambda i: (i, 0))
    wspec = lambda shape: pl.BlockSpec(shape, lambda i: (0, 0))
    return pl.pallas_call(
        _dense_body,
        grid=(BATCH // BB,),
        in_specs=[
            bspec((BB, 2 * DIM)), bspec((BB, 2 * DIM)),
            bspec((BB, 1)), bspec((BB, 1)),
            wspec((DIM, 64)), wspec((DIM, 64)), wspec((1, 64)),
            wspec((64, 32)), wspec((1, 32)),
            wspec((32, 16)), wspec((1, 16)),
            wspec((DIM, 1)), wspec((16, 1)), wspec((1, 1)),
        ],
        out_specs=pl.BlockSpec((BB, 1), lambda i: (i, 0)),
        out_shape=jax.ShapeDtypeStruct((BATCH, 1), jnp.float32),
    )(gu, gi, upar, ipar, w0u, w0i, b0, w1, b1, w2, b2, wfm, wfh, bf)


def kernel(user, item, mf_user_embed, mf_item_embed, mlp_user_embed,
           mlp_item_embed, W0, b0, W1, b1, W2, b2, Wf, bf):
    user = user.astype(jnp.int32)
    item = item.astype(jnp.int32)

    p_user = _tc_transpose_pack(mf_user_embed.T, mlp_user_embed.T)
    p_item = _tc_transpose_pack(mf_item_embed.T, mlp_item_embed.T)

    def rowmap(r):
        w = r // TW
        return (w // 2) * TW + r % TW, (w % 2).astype(jnp.float32)

    u2, upar = rowmap(user)
    i2, ipar = rowmap(item)
    gu, gi = _sc_gather2(u2.reshape(BATCH // CH, CH),
                         i2.reshape(BATCH // CH, CH), p_user, p_item)

    w0t = W0.T
    wft = Wf.T
    out = _tc_dense(
        gu, gi, upar.reshape(BATCH, 1), ipar.reshape(BATCH, 1),
        w0t[:DIM], w0t[DIM:], b0.reshape(1, -1),
        W1.T, b1.reshape(1, -1),
        W2.T, b2.reshape(1, -1),
        wft[:DIM], wft[DIM:], bf.reshape(1, 1))
    return out


# speedup vs baseline: 3.5998x; 1.0100x over previous
import functools

import jax
import jax.numpy as jnp
from jax import lax
from jax.experimental import pallas as pl
from jax.experimental.pallas import tpu as pltpu
from jax.experimental.pallas import tpu_sc as plsc

BATCH = 16384
DIM = 64
NC, NS = 2, 16
NW = NC * NS
B_PER_W = BATCH // NW
CH = 128
NCH = B_PER_W // CH

TW = 4096


def _pack2(a, b):
    au = lax.bitcast_convert_type(
        a.astype(jnp.bfloat16).T, jnp.uint16).astype(jnp.uint32)
    bu = lax.bitcast_convert_type(
        b.astype(jnp.bfloat16).T, jnp.uint16).astype(jnp.uint32)
    return (au << 16) | bu


def _xpose_body(a_ref, b_ref, o_ref):
    a = a_ref[...]
    b = b_ref[...]
    w = jnp.concatenate([_pack2(a[:, :TW], b[:, :TW]),
                         _pack2(a[:, TW:], b[:, TW:])], axis=1)
    o_ref[...] = lax.bitcast_convert_type(w, jnp.int32)


def _tc_transpose_pack(ta_T, tb_T):
    n = ta_T.shape[1]
    grid = pl.cdiv(n, 2 * TW)
    return pl.pallas_call(
        _xpose_body,
        grid=(grid,),
        in_specs=[
            pl.BlockSpec((DIM, 2 * TW), lambda i: (0, i)),
            pl.BlockSpec((DIM, 2 * TW), lambda i: (0, i)),
        ],
        out_specs=pl.BlockSpec((TW, 2 * DIM), lambda i: (i, 0)),
        out_shape=jax.ShapeDtypeStruct((grid * TW, 2 * DIM), jnp.int32),
    )(ta_T, tb_T)


def _sc_gather2(u2, i2, p_user, p_item):
    mesh = plsc.VectorSubcoreMesh(core_axis_name="c", subcore_axis_name="s")
    row_t = jax.ShapeDtypeStruct((BATCH, DIM), jnp.int32)

    @functools.partial(
        pl.kernel,
        out_type=(row_t, row_t),
        mesh=mesh,
        compiler_params=pltpu.CompilerParams(use_tc_tiling_on_sc=False),
        scratch_types=[
            pltpu.VMEM((NCH, CH), jnp.int32),
            pltpu.VMEM((NCH, CH), jnp.int32),
            pltpu.VMEM((CH, DIM), jnp.int32),
            pltpu.VMEM((CH, DIM), jnp.int32),
            pltpu.SemaphoreType.DMA,
            pltpu.SemaphoreType.DMA,
            pltpu.SemaphoreType.DMA,
            pltpu.SemaphoreType.DMA,
        ],
    )
    def k(u_hbm, i_hbm, pu_hbm, pi_hbm, o_u, o_i,
          uidx, iidx, rows_a, rows_b, sem_a, sem_b, sem_wa, sem_wb):
        wid = lax.axis_index("s") * NC + lax.axis_index("c")
        base = wid * B_PER_W

        pu2 = pu_hbm
        pi2 = pi_hbm

        pltpu.sync_copy(u_hbm.at[pl.ds(wid * NCH, NCH)], uidx)
        pltpu.sync_copy(i_hbm.at[pl.ds(wid * NCH, NCH)], iidx)

        work = []
        for table, idx, out in ((pu2, uidx, o_u), (pi2, iidx, o_i)):
            for c in range(NCH):
                work.append((table, idx, c, out))

        bufs = ((rows_a, sem_a, sem_wa), (rows_b, sem_b, sem_wb))
        pending_w = [None, None]
        for n, (table, idx, c, out) in enumerate(work):
            rows, sem_g, sem_w = bufs[n % 2]
            if pending_w[n % 2] is not None:
                pending_w[n % 2].wait()
            g = pltpu.async_copy(table.at[idx.at[c]], rows, sem_g)
            g.wait()
            pending_w[n % 2] = pltpu.async_copy(
                rows, out.at[pl.ds(base + c * CH, CH)], sem_w)
        for w in pending_w:
            if w is not None:
                w.wait()

    return k(u2, i2, p_user, p_item)


def _mm(a, b):
    return lax.dot_general(a, b, (((1,), (0,)), ((), ())),
                           preferred_element_type=jnp.float32)


BB = 2048

def _unpack(words):
    u = lax.bitcast_convert_type(words, jnp.uint32)
    mf = lax.bitcast_convert_type((u >> 16) << 16, jnp.float32)
    ml = lax.bitcast_convert_type(u << 16, jnp.float32)
    return mf, ml


def _dense_body(gu, gi,
                w0u, w0i, b0, w1, b1, w2, b2, wfm, wfh, bf, o):
    xmfu, xmlu = _unpack(gu[...])
    xmfi, xmli = _unpack(gi[...])

    h = _mm(xmlu, w0u[...]) + _mm(xmli, w0i[...]) + b0[...]
    h = jnp.maximum(h, 0.0)
    h = jnp.maximum(_mm(h, w1[...]) + b1[...], 0.0)
    h = jnp.maximum(_mm(h, w2[...]) + b2[...], 0.0)
    xmf = xmfu * xmfi
    o[...] = _mm(xmf, wfm[...]) + _mm(h, wfh[...]) + bf[...]


def _tc_dense(gu, gi, w0u, w0i, b0, w1, b1, w2, b2, wfm, wfh, bf):
    bspec = lambda shape: pl.BlockSpec(shape, lambda i: (i, 0))
    wspec = lambda shape: pl.BlockSpec(shape, lambda i: (0, 0))
    return pl.pallas_call(
        _dense_body,
        grid=(BATCH // BB,),
        in_specs=[
            bspec((BB, DIM)), bspec((BB, DIM)),
            wspec((DIM, 64)), wspec((DIM, 64)), wspec((1, 64)),
            wspec((64, 32)), wspec((1, 32)),
            wspec((32, 16)), wspec((1, 16)),
            wspec((DIM, 1)), wspec((16, 1)), wspec((1, 1)),
        ],
        out_specs=pl.BlockSpec((BB, 1), lambda i: (i, 0)),
        out_shape=jax.ShapeDtypeStruct((BATCH, 1), jnp.float32),
    )(gu, gi, w0u, w0i, b0, w1, b1, w2, b2, wfm, wfh, bf)


def kernel(user, item, mf_user_embed, mf_item_embed, mlp_user_embed,
           mlp_item_embed, W0, b0, W1, b1, W2, b2, Wf, bf):
    user = user.astype(jnp.int32)
    item = item.astype(jnp.int32)

    p_user = _tc_transpose_pack(mf_user_embed.T, mlp_user_embed.T)
    p_item = _tc_transpose_pack(mf_item_embed.T, mlp_item_embed.T)

    def rowmap(r):
        w = r // TW
        return (w // 2) * (2 * TW) + 2 * (r % TW) + (w % 2)

    u2 = rowmap(user)
    i2 = rowmap(item)
    gu, gi = _sc_gather2(u2.reshape(BATCH // CH, CH),
                         i2.reshape(BATCH // CH, CH),
                         p_user.reshape(-1, DIM), p_item.reshape(-1, DIM))

    w0t = W0.T
    wft = Wf.T
    out = _tc_dense(
        gu, gi,
        w0t[:DIM], w0t[DIM:], b0.reshape(1, -1),
        W1.T, b1.reshape(1, -1),
        W2.T, b2.reshape(1, -1),
        wft[:DIM], wft[DIM:], bf.reshape(1, 1))
    return out


# speedup vs baseline: 3.9468x; 1.0964x over previous
import functools

import jax
import jax.numpy as jnp
from jax import lax
from jax.experimental import pallas as pl
from jax.experimental.pallas import tpu as pltpu
from jax.experimental.pallas import tpu_sc as plsc

BATCH = 16384
DIM = 64
NC, NS = 2, 16
NW = NC * NS
B_PER_W = BATCH // NW
CH = 128
NCH = B_PER_W // CH

TW = 8192


def _pack2(a, b):
    au = lax.bitcast_convert_type(
        a.astype(jnp.bfloat16).T, jnp.uint16).astype(jnp.uint32)
    bu = lax.bitcast_convert_type(
        b.astype(jnp.bfloat16).T, jnp.uint16).astype(jnp.uint32)
    return (au << 16) | bu


def _xpose_body(a_ref, b_ref, o_ref):
    a = a_ref[...]
    b = b_ref[...]
    w = jnp.concatenate([_pack2(a[:, :TW], b[:, :TW]),
                         _pack2(a[:, TW:], b[:, TW:])], axis=1)
    o_ref[...] = lax.bitcast_convert_type(w, jnp.int32)


def _tc_transpose_pack(ta_T, tb_T):
    n = ta_T.shape[1]
    grid = pl.cdiv(n, 2 * TW)
    return pl.pallas_call(
        _xpose_body,
        grid=(grid,),
        in_specs=[
            pl.BlockSpec((DIM, 2 * TW), lambda i: (0, i)),
            pl.BlockSpec((DIM, 2 * TW), lambda i: (0, i)),
        ],
        out_specs=pl.BlockSpec((TW, 2 * DIM), lambda i: (i, 0)),
        out_shape=jax.ShapeDtypeStruct((grid * TW, 2 * DIM), jnp.int32),
    )(ta_T, tb_T)


def _sc_gather2(u2, i2, p_user, p_item):
    mesh = plsc.VectorSubcoreMesh(core_axis_name="c", subcore_axis_name="s")
    row_t = jax.ShapeDtypeStruct((BATCH, DIM), jnp.int32)

    @functools.partial(
        pl.kernel,
        out_type=(row_t, row_t),
        mesh=mesh,
        compiler_params=pltpu.CompilerParams(use_tc_tiling_on_sc=False),
        scratch_types=[
            pltpu.VMEM((NCH, CH), jnp.int32),
            pltpu.VMEM((NCH, CH), jnp.int32),
            pltpu.VMEM((CH, DIM), jnp.int32),
            pltpu.VMEM((CH, DIM), jnp.int32),
            pltpu.SemaphoreType.DMA,
            pltpu.SemaphoreType.DMA,
            pltpu.SemaphoreType.DMA,
            pltpu.SemaphoreType.DMA,
        ],
    )
    def k(u_hbm, i_hbm, pu_hbm, pi_hbm, o_u, o_i,
          uidx, iidx, rows_a, rows_b, sem_a, sem_b, sem_wa, sem_wb):
        wid = lax.axis_index("s") * NC + lax.axis_index("c")
        base = wid * B_PER_W

        pu2 = pu_hbm
        pi2 = pi_hbm

        pltpu.sync_copy(u_hbm.at[pl.ds(wid * NCH, NCH)], uidx)
        pltpu.sync_copy(i_hbm.at[pl.ds(wid * NCH, NCH)], iidx)

        work = []
        for table, idx, out in ((pu2, uidx, o_u), (pi2, iidx, o_i)):
            for c in range(NCH):
                work.append((table, idx, c, out))

        bufs = ((rows_a, sem_a, sem_wa), (rows_b, sem_b, sem_wb))
        pending_w = [None, None]
        for n, (table, idx, c, out) in enumerate(work):
            rows, sem_g, sem_w = bufs[n % 2]
            if pending_w[n % 2] is not None:
                pending_w[n % 2].wait()
            g = pltpu.async_copy(table.at[idx.at[c]], rows, sem_g)
            g.wait()
            pending_w[n % 2] = pltpu.async_copy(
                rows, out.at[pl.ds(base + c * CH, CH)], sem_w)
        for w in pending_w:
            if w is not None:
                w.wait()

    return k(u2, i2, p_user, p_item)


def _mm(a, b):
    return lax.dot_general(a, b, (((1,), (0,)), ((), ())),
                           preferred_element_type=jnp.float32)


BB = 2048

def _unpack(words):
    u = lax.bitcast_convert_type(words, jnp.uint32)
    mf = lax.bitcast_convert_type((u >> 16) << 16, jnp.float32)
    ml = lax.bitcast_convert_type(u << 16, jnp.float32)
    return mf, ml


def _dense_body(gu, gi,
                w0u, w0i, b0, w1, b1, w2, b2, wfm, wfh, bf, o):
    xmfu, xmlu = _unpack(gu[...])
    xmfi, xmli = _unpack(gi[...])

    h = _mm(xmlu, w0u[...]) + _mm(xmli, w0i[...]) + b0[...]
    h = jnp.maximum(h, 0.0)
    h = jnp.maximum(_mm(h, w1[...]) + b1[...], 0.0)
    h = jnp.maximum(_mm(h, w2[...]) + b2[...], 0.0)
    xmf = xmfu * xmfi
    o[...] = _mm(xmf, wfm[...]) + _mm(h, wfh[...]) + bf[...]


def _tc_dense(gu, gi, w0u, w0i, b0, w1, b1, w2, b2, wfm, wfh, bf):
    bspec = lambda shape: pl.BlockSpec(shape, lambda i: (i, 0))
    wspec = lambda shape: pl.BlockSpec(shape, lambda i: (0, 0))
    return pl.pallas_call(
        _dense_body,
        grid=(BATCH // BB,),
        in_specs=[
            bspec((BB, DIM)), bspec((BB, DIM)),
            wspec((DIM, 64)), wspec((DIM, 64)), wspec((1, 64)),
            wspec((64, 32)), wspec((1, 32)),
            wspec((32, 16)), wspec((1, 16)),
            wspec((DIM, 1)), wspec((16, 1)), wspec((1, 1)),
        ],
        out_specs=pl.BlockSpec((BB, 1), lambda i: (i, 0)),
        out_shape=jax.ShapeDtypeStruct((BATCH, 1), jnp.float32),
    )(gu, gi, w0u, w0i, b0, w1, b1, w2, b2, wfm, wfh, bf)


def kernel(user, item, mf_user_embed, mf_item_embed, mlp_user_embed,
           mlp_item_embed, W0, b0, W1, b1, W2, b2, Wf, bf):
    user = user.astype(jnp.int32)
    item = item.astype(jnp.int32)

    p_user = _tc_transpose_pack(mf_user_embed.T, mlp_user_embed.T)
    p_item = _tc_transpose_pack(mf_item_embed.T, mlp_item_embed.T)

    def rowmap(r):
        w = r // TW
        return (w // 2) * (2 * TW) + 2 * (r % TW) + (w % 2)

    u2 = rowmap(user)
    i2 = rowmap(item)
    gu, gi = _sc_gather2(u2.reshape(BATCH // CH, CH),
                         i2.reshape(BATCH // CH, CH),
                         p_user.reshape(-1, DIM), p_item.reshape(-1, DIM))

    w0t = W0.T
    wft = Wf.T
    out = _tc_dense(
        gu, gi,
        w0t[:DIM], w0t[DIM:], b0.reshape(1, -1),
        W1.T, b1.reshape(1, -1),
        W2.T, b2.reshape(1, -1),
        wft[:DIM], wft[DIM:], bf.reshape(1, 1))
    return out
